# baseline (speedup 1.0000x reference)
import functools

import jax
import jax.numpy as jnp
from jax import lax
from jax.experimental import pallas as pl
from jax.experimental.pallas import tpu as pltpu
from jax.experimental.pallas import tpu_sc as plsc

N = 10000
E = 320000
D_IN = 128
HID = 128
NCLS = 64
K_HOPS = 10
ALPHA = 0.1

NC = 2
NS = 16
NW = NC * NS
CHUNK = 128
CH = 80
EPAD = NW * CH * CHUNK
DUMMY_DST = N
NP = 10240
RPT = NP // NS
RB = 1280
NBUF = 8
W = NBUF // 2
NWAVE = CH // W
RCH = RPT // CHUNK
LANE = 16

_mesh = plsc.VectorSubcoreMesh(
    core_axis_name="c", subcore_axis_name="s", num_cores=NC, num_subcores=NS
)
_sc_params = pltpu.CompilerParams(use_tc_tiling_on_sc=False)

_f32 = jnp.float32
_i32 = jnp.int32


def _mlp_body(x_ref, w0_ref, b0_ref, w1_ref, b1_ref, w2_ref, b2_ref, o_ref):
    h = jnp.dot(x_ref[...], w0_ref[...], preferred_element_type=_f32)
    h = jnp.maximum(h + b0_ref[...], 0.0)
    h = jnp.dot(h, w1_ref[...], preferred_element_type=_f32)
    h = jnp.maximum(h + b1_ref[...], 0.0)
    o_ref[...] = jnp.dot(h, w2_ref[...], preferred_element_type=_f32) + b2_ref[...]


def _mlp_tc(xp, W0, b0, W1, b1, W2, b2):
    full = lambda r, c: pl.BlockSpec((r, c), lambda i: (0, 0))
    return pl.pallas_call(
        _mlp_body,
        grid=(NP // RB,),
        in_specs=[
            pl.BlockSpec((RB, D_IN), lambda i: (i, 0)),
            full(D_IN, HID), full(1, HID),
            full(HID, HID), full(1, HID),
            full(HID, NCLS), full(1, NCLS),
        ],
        out_specs=pl.BlockSpec((RB, NCLS), lambda i: (i, 0)),
        out_shape=jax.ShapeDtypeStruct((NP, NCLS), _f32),
    )(xp, W0, b0.reshape(1, HID), W1, b1.reshape(1, HID), W2, b2.reshape(1, NCLS))


@functools.partial(
    pl.kernel,
    out_type=(
        jax.ShapeDtypeStruct((NP, NCLS), _f32),
        jax.ShapeDtypeStruct((NC * NP, NCLS), _f32),
        jax.ShapeDtypeStruct((NC, NP, NCLS), _f32),
        jax.ShapeDtypeStruct((3 * NC * NP, NCLS), _f32),
        jax.ShapeDtypeStruct((NC, 16, LANE), _i32),
    ),
    mesh=_mesh,
    scratch_types=[
        pltpu.VMEM((CH, CHUNK), _i32),
        pltpu.VMEM((CH, CHUNK), _i32),
        [pltpu.VMEM((CHUNK, NCLS), _f32)] * NBUF,
        pltpu.VMEM((LANE,), _i32),
        pltpu.VMEM_SHARED((NP, NCLS), _f32),
        [pltpu.SemaphoreType.DMA] * NBUF,
        [pltpu.SemaphoreType.DMA] * NBUF,
    ],
    compiler_params=_sc_params,
)
def _appnp_sc(h0_hbm, srcp_hbm, dstp_hbm, zeros_hbm, ones_hbm, onesf_hbm,
              zflag_hbm, hout, feat, pex, aux, flags,
              idx_s, idx_d, bufs, fbuf, agg, gsems, ssems):
    c = lax.axis_index("c")
    s = lax.axis_index("s")
    w = c * NS + s
    oc = 1 - c
    base = s * RPT

    pltpu.sync_copy(srcp_hbm.at[w], idx_s)
    pltpu.sync_copy(dstp_hbm.at[w], idx_d)

    @pl.when(s == 0)
    def _():
        pltpu.sync_copy(zflag_hbm, flags.at[c])

    off = c * NP

    def offs_body(r, carry):
        for co in range(CHUNK // LANE):
            sl = pl.ds(co * LANE, LANE)
            idx_s[r, sl] = idx_s[r, sl] + off
        return carry

    lax.fori_loop(0, CH, offs_body, 0)

    def zero_agg():
        pltpu.sync_copy(zeros_hbm, agg.at[pl.ds(base, RPT)])

    def publish_and_wait(slot):
        pltpu.sync_copy(agg.at[pl.ds(base, RPT)], pex.at[c, pl.ds(base, RPT)])
        plsc.subcore_barrier()

        @pl.when(s == 0)
        def _():
            pltpu.sync_copy(onesf_hbm, flags.at[c, slot])

        fbuf[...] = jnp.zeros((LANE,), _i32)

        def outer(i, carry):
            v = fbuf[...]

            @pl.when(v[0] < 1)
            def _():
                def inner(j, c2):
                    v2 = fbuf[...]

                    @pl.when(v2[0] < 1)
                    def _():
                        pltpu.sync_copy(flags.at[oc, slot], fbuf)

                    return c2

                lax.fori_loop(0, 64, inner, 0)

            return carry

        lax.fori_loop(0, 32, outer, 0)

    def edges_phase():
        def wavepair(g, carry):
            for sub in range(2):
                wv = 2 * g + sub
                half = sub * W
                ohalf = (1 - sub) * W

                @pl.when(wv >= 2)
                def _():
                    for i in range(W):
                        pltpu.make_async_copy(
                            bufs[half + i], agg.at[idx_d.at[(wv - 2) * W + i]],
                            ssems[half + i],
                        ).wait()

                for i in range(W):
                    pltpu.async_copy(
                        feat.at[idx_s.at[wv * W + i]], bufs[half + i],
                        gsems[half + i],
                    )

                @pl.when(wv >= 1)
                def _():
                    for i in range(W):
                        pltpu.make_async_copy(
                            feat.at[idx_s.at[(wv - 1) * W + i]], bufs[ohalf + i],
                            gsems[ohalf + i],
                        ).wait()
                    for i in range(W):
                        pltpu.async_copy(
                            bufs[ohalf + i], agg.at[idx_d.at[(wv - 1) * W + i]],
                            ssems[ohalf + i], add=True,
                        )
            return carry

        lax.fori_loop(0, NWAVE // 2, wavepair, 0)
        lhalf = ((NWAVE - 1) % 2) * W
        phalf = (NWAVE % 2) * W
        for i in range(W):
            pltpu.make_async_copy(
                feat.at[idx_s.at[(NWAVE - 1) * W + i]], bufs[lhalf + i],
                gsems[lhalf + i],
            ).wait()
        for i in range(W):
            pltpu.async_copy(
                bufs[lhalf + i], agg.at[idx_d.at[(NWAVE - 1) * W + i]],
                ssems[lhalf + i], add=True,
            )
        for i in range(W):
            pltpu.make_async_copy(
                bufs[phalf + i], agg.at[idx_d.at[(NWAVE - 2) * W + i]],
                ssems[phalf + i],
            ).wait()
        for i in range(W):
            pltpu.make_async_copy(
                bufs[lhalf + i], agg.at[idx_d.at[(NWAVE - 1) * W + i]],
                ssems[lhalf + i],
            ).wait()

    zero_agg()
    pltpu.sync_copy(ones_hbm, bufs[0])
    plsc.subcore_barrier()

    def deg_body(g, carry):
        for b in range(NBUF):
            pltpu.async_copy(
                bufs[0], agg.at[idx_d.at[g * NBUF + b]], ssems[b], add=True
            )
        for b in range(NBUF):
            pltpu.make_async_copy(
                bufs[0], agg.at[idx_d.at[g * NBUF + b]], ssems[b]
            ).wait()
        return carry

    lax.fori_loop(0, CH // NBUF, deg_body, 0)
    plsc.subcore_barrier()
    publish_and_wait(0)

    n2_off = (3 * c + 0) * NP
    f0_off = (3 * c + 1) * NP
    nb_off = (3 * c + 2) * NP
    magic = jnp.full((LANE,), 0x5F3759DF, _i32)

    for rc in range(RCH):
        rb = base + rc * CHUNK
        pltpu.sync_copy(pex.at[0, pl.ds(rb, CHUNK)], bufs[0])
        pltpu.sync_copy(pex.at[1, pl.ds(rb, CHUNK)], bufs[1])
        pltpu.sync_copy(h0_hbm.at[pl.ds(rb, CHUNK)], bufs[2])

        def norm_body(r, carry):
            for co in range(NCLS // LANE):
                sl = pl.ds(co * LANE, LANE)
                d = jnp.maximum(bufs[0][r, sl] + bufs[1][r, sl], 1.0)
                i = magic - lax.shift_right_logical(
                    lax.bitcast_convert_type(d, _i32), 1
                )
                y = lax.bitcast_convert_type(i, _f32)
                y = y * (1.5 - 0.5 * d * y * y)
                y = y * (1.5 - 0.5 * d * y * y)
                y = y * (1.5 - 0.5 * d * y * y)
                bufs[3][r, sl] = y * y
                f0v = bufs[2][r, sl] * y
                bufs[4][r, sl] = f0v
                bufs[5][r, sl] = y
            return carry

        lax.fori_loop(0, CHUNK, norm_body, 0)
        pltpu.sync_copy(bufs[3], aux.at[pl.ds(n2_off + rb, CHUNK)])
        pltpu.sync_copy(bufs[4], aux.at[pl.ds(f0_off + rb, CHUNK)])
        pltpu.sync_copy(bufs[5], aux.at[pl.ds(nb_off + rb, CHUNK)])
        pltpu.sync_copy(bufs[4], feat.at[pl.ds(c * NP + rb, CHUNK)])

    def combine_to_feat():
        for rc in range(RCH):
            rb = base + rc * CHUNK
            pltpu.sync_copy(pex.at[0, pl.ds(rb, CHUNK)], bufs[0])
            pltpu.sync_copy(pex.at[1, pl.ds(rb, CHUNK)], bufs[1])
            pltpu.sync_copy(aux.at[pl.ds(n2_off + rb, CHUNK)], bufs[2])
            pltpu.sync_copy(aux.at[pl.ds(f0_off + rb, CHUNK)], bufs[3])

            def cbody(r, carry):
                for co in range(NCLS // LANE):
                    sl = pl.ds(co * LANE, LANE)
                    bufs[4][r, sl] = (
                        (1.0 - ALPHA) * bufs[2][r, sl]
                        * (bufs[0][r, sl] + bufs[1][r, sl])
                        + ALPHA * bufs[3][r, sl]
                    )
                return carry

            lax.fori_loop(0, CHUNK, cbody, 0)
            pltpu.sync_copy(bufs[4], feat.at[pl.ds(c * NP + rb, CHUNK)])

    def hop_body(k, carry):
        with jax.named_scope("zero"):
            zero_agg()
            plsc.subcore_barrier()
        with jax.named_scope("edges"):
            edges_phase()
            plsc.subcore_barrier()
        with jax.named_scope("exchange"):
            publish_and_wait(1 + k)
        with jax.named_scope("combine"):
            combine_to_feat()
        return carry

    lax.fori_loop(0, K_HOPS - 1, hop_body, 0)

    zero_agg()
    plsc.subcore_barrier()
    edges_phase()
    plsc.subcore_barrier()
    publish_and_wait(K_HOPS)

    @pl.when(c == 0)
    def _():
        for rc in range(RCH):
            rb = base + rc * CHUNK
            pltpu.sync_copy(pex.at[0, pl.ds(rb, CHUNK)], bufs[0])
            pltpu.sync_copy(pex.at[1, pl.ds(rb, CHUNK)], bufs[1])
            pltpu.sync_copy(aux.at[pl.ds(nb_off + rb, CHUNK)], bufs[2])
            pltpu.sync_copy(h0_hbm.at[pl.ds(rb, CHUNK)], bufs[3])

            def fbody(r, carry):
                for co in range(NCLS // LANE):
                    sl = pl.ds(co * LANE, LANE)
                    bufs[4][r, sl] = (
                        (1.0 - ALPHA) * bufs[2][r, sl]
                        * (bufs[0][r, sl] + bufs[1][r, sl])
                        + ALPHA * bufs[3][r, sl]
                    )
                return carry

            lax.fori_loop(0, CHUNK, fbody, 0)
            pltpu.sync_copy(bufs[4], hout.at[pl.ds(rb, CHUNK)])


def kernel(features, edge_index, W0, b0, W1, b1, W2, b2):
    src = edge_index[0]
    dst = edge_index[1]
    pad = EPAD - E
    srcp = jnp.concatenate([src, jnp.zeros((pad,), _i32)]).reshape(NW, CH, CHUNK)
    dstp = jnp.concatenate(
        [dst, jnp.full((pad,), DUMMY_DST, _i32)]
    ).reshape(NW, CH, CHUNK)

    xp = jnp.pad(features, ((0, NP - N), (0, 0)))
    h0p = _mlp_tc(xp, W0, b0, W1, b1, W2, b2)

    zeros64 = jnp.zeros((RPT, NCLS), _f32)
    ones64 = jnp.ones((CHUNK, NCLS), _f32)
    onesf = jnp.ones((LANE,), _i32)
    zflag = jnp.zeros((16, LANE), _i32)

    hout, _, _, _, _ = _appnp_sc(h0p, srcp, dstp, zeros64, ones64, onesf, zflag)
    return hout[:N]


# speedup vs baseline: 1.0601x; 1.0601x over previous
import functools

import jax
import jax.numpy as jnp
from jax import lax
from jax.experimental import pallas as pl
from jax.experimental.pallas import tpu as pltpu
from jax.experimental.pallas import tpu_sc as plsc

N = 10000
E = 320000
D_IN = 128
HID = 128
NCLS = 64
K_HOPS = 10
ALPHA = 0.1

NC = 2
NS = 16
NW = NC * NS
CHUNK = 128
CH = 80
EPAD = NW * CH * CHUNK
DUMMY_DST = N
NP = 10240
RPT = NP // NS
RB = 1280
NBUF = 8
W = NBUF // 2
NWAVE = CH // W
RCH = RPT // CHUNK
LANE = 16

_mesh = plsc.VectorSubcoreMesh(
    core_axis_name="c", subcore_axis_name="s", num_cores=NC, num_subcores=NS
)
_sc_params = pltpu.CompilerParams(use_tc_tiling_on_sc=False)

_f32 = jnp.float32
_i32 = jnp.int32


def _mlp_body(x_ref, w0_ref, b0_ref, w1_ref, b1_ref, w2_ref, b2_ref, o_ref):
    h = jnp.dot(x_ref[...], w0_ref[...], preferred_element_type=_f32)
    h = jnp.maximum(h + b0_ref[...], 0.0)
    h = jnp.dot(h, w1_ref[...], preferred_element_type=_f32)
    h = jnp.maximum(h + b1_ref[...], 0.0)
    o_ref[...] = jnp.dot(h, w2_ref[...], preferred_element_type=_f32) + b2_ref[...]


def _mlp_tc(xp, W0, b0, W1, b1, W2, b2):
    full = lambda r, c: pl.BlockSpec((r, c), lambda i: (0, 0))
    return pl.pallas_call(
        _mlp_body,
        grid=(NP // RB,),
        in_specs=[
            pl.BlockSpec((RB, D_IN), lambda i: (i, 0)),
            full(D_IN, HID), full(1, HID),
            full(HID, HID), full(1, HID),
            full(HID, NCLS), full(1, NCLS),
        ],
        out_specs=pl.BlockSpec((RB, NCLS), lambda i: (i, 0)),
        out_shape=jax.ShapeDtypeStruct((NP, NCLS), _f32),
    )(xp, W0, b0.reshape(1, HID), W1, b1.reshape(1, HID), W2, b2.reshape(1, NCLS))


@functools.partial(
    pl.kernel,
    out_type=(
        jax.ShapeDtypeStruct((NP, NCLS), _f32),
        jax.ShapeDtypeStruct((NC * NP, NCLS), _f32),
        jax.ShapeDtypeStruct((NC, NP, NCLS), _f32),
        jax.ShapeDtypeStruct((3 * NC * NP, NCLS), _f32),
        jax.ShapeDtypeStruct((NC, 16, LANE), _i32),
    ),
    mesh=_mesh,
    scratch_types=[
        pltpu.VMEM((CH, CHUNK), _i32),
        pltpu.VMEM((CH, CHUNK), _i32),
        [pltpu.VMEM((CHUNK, NCLS), _f32)] * NBUF,
        pltpu.VMEM((LANE,), _i32),
        pltpu.VMEM_SHARED((NP, NCLS), _f32),
        [pltpu.SemaphoreType.DMA] * NBUF,
        [pltpu.SemaphoreType.DMA] * NBUF,
    ],
    compiler_params=_sc_params,
)
def _appnp_sc(h0_hbm, srcp_hbm, dstp_hbm, zeros_hbm, ones_hbm, onesf_hbm,
              zflag_hbm, hout, feat, pex, aux, flags,
              idx_s, idx_d, bufs, fbuf, agg, gsems, ssems):
    c = lax.axis_index("c")
    s = lax.axis_index("s")
    w = c * NS + s
    oc = 1 - c
    base = s * RPT

    pltpu.sync_copy(srcp_hbm.at[w], idx_s)
    pltpu.sync_copy(dstp_hbm.at[w], idx_d)

    @pl.when(s == 0)
    def _():
        pltpu.sync_copy(zflag_hbm, flags.at[c])

    off = c * NP

    def offs_body(r, carry):
        for co in range(CHUNK // LANE):
            sl = pl.ds(co * LANE, LANE)
            idx_s[r, sl] = idx_s[r, sl] + off
        return carry

    lax.fori_loop(0, CH, offs_body, 0)

    def zero_agg():
        pltpu.sync_copy(zeros_hbm, agg.at[pl.ds(base, RPT)])

    def publish_and_wait(slot):
        pltpu.sync_copy(agg.at[pl.ds(base, RPT)], pex.at[c, pl.ds(base, RPT)])
        plsc.subcore_barrier()

        @pl.when(s == 0)
        def _():
            pltpu.sync_copy(onesf_hbm, flags.at[c, slot])

        fbuf[...] = jnp.zeros((LANE,), _i32)

        def outer(i, carry):
            v = fbuf[...]

            @pl.when(v[0] < 1)
            def _():
                def inner(j, c2):
                    v2 = fbuf[...]

                    @pl.when(v2[0] < 1)
                    def _():
                        pltpu.sync_copy(flags.at[oc, slot], fbuf)

                    return c2

                lax.fori_loop(0, 64, inner, 0)

            return carry

        lax.fori_loop(0, 32, outer, 0)

    def edges_phase():
        def wavepair(g, carry):
            for sub in range(2):
                wv = 2 * g + sub
                half = sub * W
                ohalf = (1 - sub) * W

                @pl.when(wv >= 2)
                def _():
                    for i in range(W):
                        pltpu.make_async_copy(
                            bufs[half + i], agg.at[idx_d.at[(wv - 2) * W + i]],
                            ssems[half + i],
                        ).wait()

                for i in range(W):
                    pltpu.async_copy(
                        feat.at[idx_s.at[wv * W + i]], bufs[half + i],
                        gsems[half + i],
                    )

                @pl.when(wv >= 1)
                def _():
                    for i in range(W):
                        pltpu.make_async_copy(
                            feat.at[idx_s.at[(wv - 1) * W + i]], bufs[ohalf + i],
                            gsems[ohalf + i],
                        ).wait()
                    for i in range(W):
                        pltpu.async_copy(
                            bufs[ohalf + i], agg.at[idx_d.at[(wv - 1) * W + i]],
                            ssems[ohalf + i], add=True,
                        )
            return carry

        lax.fori_loop(0, NWAVE // 2, wavepair, 0)
        lhalf = ((NWAVE - 1) % 2) * W
        phalf = (NWAVE % 2) * W
        for i in range(W):
            pltpu.make_async_copy(
                feat.at[idx_s.at[(NWAVE - 1) * W + i]], bufs[lhalf + i],
                gsems[lhalf + i],
            ).wait()
        for i in range(W):
            pltpu.async_copy(
                bufs[lhalf + i], agg.at[idx_d.at[(NWAVE - 1) * W + i]],
                ssems[lhalf + i], add=True,
            )
        for i in range(W):
            pltpu.make_async_copy(
                bufs[phalf + i], agg.at[idx_d.at[(NWAVE - 2) * W + i]],
                ssems[phalf + i],
            ).wait()
        for i in range(W):
            pltpu.make_async_copy(
                bufs[lhalf + i], agg.at[idx_d.at[(NWAVE - 1) * W + i]],
                ssems[lhalf + i],
            ).wait()

    zero_agg()
    pltpu.sync_copy(ones_hbm, bufs[0])
    plsc.subcore_barrier()

    def deg_body(g, carry):
        for b in range(NBUF):
            pltpu.async_copy(
                bufs[0], agg.at[idx_d.at[g * NBUF + b]], ssems[b], add=True
            )
        for b in range(NBUF):
            pltpu.make_async_copy(
                bufs[0], agg.at[idx_d.at[g * NBUF + b]], ssems[b]
            ).wait()
        return carry

    lax.fori_loop(0, CH // NBUF, deg_body, 0)
    plsc.subcore_barrier()
    publish_and_wait(0)

    n2_off = (3 * c + 0) * NP
    f0_off = (3 * c + 1) * NP
    nb_off = (3 * c + 2) * NP
    magic = jnp.full((LANE,), 0x5F3759DF, _i32)

    for rc in range(RCH):
        rb = base + rc * CHUNK
        pltpu.sync_copy(pex.at[0, pl.ds(rb, CHUNK)], bufs[0])
        pltpu.sync_copy(pex.at[1, pl.ds(rb, CHUNK)], bufs[1])
        pltpu.sync_copy(h0_hbm.at[pl.ds(rb, CHUNK)], bufs[2])

        def norm_body(r, carry):
            for co in range(NCLS // LANE):
                sl = pl.ds(co * LANE, LANE)
                d = jnp.maximum(bufs[0][r, sl] + bufs[1][r, sl], 1.0)
                i = magic - lax.shift_right_logical(
                    lax.bitcast_convert_type(d, _i32), 1
                )
                y = lax.bitcast_convert_type(i, _f32)
                y = y * (1.5 - 0.5 * d * y * y)
                y = y * (1.5 - 0.5 * d * y * y)
                y = y * (1.5 - 0.5 * d * y * y)
                bufs[3][r, sl] = y * y
                f0v = bufs[2][r, sl] * y
                bufs[4][r, sl] = f0v
                bufs[5][r, sl] = y
            return carry

        lax.fori_loop(0, CHUNK, norm_body, 0)
        pltpu.sync_copy(bufs[3], aux.at[pl.ds(n2_off + rb, CHUNK)])
        pltpu.sync_copy(bufs[4], aux.at[pl.ds(f0_off + rb, CHUNK)])
        pltpu.sync_copy(bufs[5], aux.at[pl.ds(nb_off + rb, CHUNK)])
        pltpu.sync_copy(bufs[4], feat.at[pl.ds(c * NP + rb, CHUNK)])

    def _comb_fire_loads(rc, hb):
        rb = base + rc * CHUNK
        pltpu.async_copy(pex.at[0, pl.ds(rb, CHUNK)], bufs[hb + 0], gsems[hb + 0])
        pltpu.async_copy(pex.at[1, pl.ds(rb, CHUNK)], bufs[hb + 1], gsems[hb + 1])
        pltpu.async_copy(
            aux.at[pl.ds(n2_off + rb, CHUNK)], bufs[hb + 2], gsems[hb + 2]
        )
        pltpu.async_copy(
            aux.at[pl.ds(f0_off + rb, CHUNK)], bufs[hb + 3], gsems[hb + 3]
        )

    def _comb_wait_loads(rc, hb):
        rb = base + rc * CHUNK
        pltpu.make_async_copy(
            pex.at[0, pl.ds(rb, CHUNK)], bufs[hb + 0], gsems[hb + 0]
        ).wait()
        pltpu.make_async_copy(
            pex.at[1, pl.ds(rb, CHUNK)], bufs[hb + 1], gsems[hb + 1]
        ).wait()
        pltpu.make_async_copy(
            aux.at[pl.ds(n2_off + rb, CHUNK)], bufs[hb + 2], gsems[hb + 2]
        ).wait()
        pltpu.make_async_copy(
            aux.at[pl.ds(f0_off + rb, CHUNK)], bufs[hb + 3], gsems[hb + 3]
        ).wait()

    def combine_to_feat():
        _comb_fire_loads(0, 0)
        for rc in range(RCH):
            hb = 4 * (rc % 2)
            if rc + 1 < RCH:
                if rc >= 1:
                    prb = base + (rc - 1) * CHUNK
                    pltpu.make_async_copy(
                        bufs[4 * ((rc - 1) % 2)],
                        feat.at[pl.ds(c * NP + prb, CHUNK)],
                        ssems[(rc - 1) % 2],
                    ).wait()
                _comb_fire_loads(rc + 1, 4 * ((rc + 1) % 2))
            _comb_wait_loads(rc, hb)

            def cbody(r, carry):
                for co in range(NCLS // LANE):
                    sl = pl.ds(co * LANE, LANE)
                    bufs[hb][r, sl] = (
                        (1.0 - ALPHA) * bufs[hb + 2][r, sl]
                        * (bufs[hb][r, sl] + bufs[hb + 1][r, sl])
                        + ALPHA * bufs[hb + 3][r, sl]
                    )
                return carry

            lax.fori_loop(0, CHUNK, cbody, 0)
            rb = base + rc * CHUNK
            pltpu.async_copy(
                bufs[hb], feat.at[pl.ds(c * NP + rb, CHUNK)], ssems[rc % 2]
            )
        for rc in (RCH - 2, RCH - 1):
            rb = base + rc * CHUNK
            pltpu.make_async_copy(
                bufs[4 * (rc % 2)], feat.at[pl.ds(c * NP + rb, CHUNK)],
                ssems[rc % 2],
            ).wait()

    def hop_body(k, carry):
        with jax.named_scope("zero"):
            zero_agg()
            plsc.subcore_barrier()
        with jax.named_scope("edges"):
            edges_phase()
            plsc.subcore_barrier()
        with jax.named_scope("exchange"):
            publish_and_wait(1 + k)
        with jax.named_scope("combine"):
            combine_to_feat()
        return carry

    lax.fori_loop(0, K_HOPS - 1, hop_body, 0)

    zero_agg()
    plsc.subcore_barrier()
    edges_phase()
    plsc.subcore_barrier()
    publish_and_wait(K_HOPS)

    @pl.when(c == 0)
    def _():
        for rc in range(RCH):
            rb = base + rc * CHUNK
            pltpu.sync_copy(pex.at[0, pl.ds(rb, CHUNK)], bufs[0])
            pltpu.sync_copy(pex.at[1, pl.ds(rb, CHUNK)], bufs[1])
            pltpu.sync_copy(aux.at[pl.ds(nb_off + rb, CHUNK)], bufs[2])
            pltpu.sync_copy(h0_hbm.at[pl.ds(rb, CHUNK)], bufs[3])

            def fbody(r, carry):
                for co in range(NCLS // LANE):
                    sl = pl.ds(co * LANE, LANE)
                    bufs[4][r, sl] = (
                        (1.0 - ALPHA) * bufs[2][r, sl]
                        * (bufs[0][r, sl] + bufs[1][r, sl])
                        + ALPHA * bufs[3][r, sl]
                    )
                return carry

            lax.fori_loop(0, CHUNK, fbody, 0)
            pltpu.sync_copy(bufs[4], hout.at[pl.ds(rb, CHUNK)])


def kernel(features, edge_index, W0, b0, W1, b1, W2, b2):
    src = edge_index[0]
    dst = edge_index[1]
    pad = EPAD - E
    srcp = jnp.concatenate([src, jnp.zeros((pad,), _i32)]).reshape(NW, CH, CHUNK)
    dstp = jnp.concatenate(
        [dst, jnp.full((pad,), DUMMY_DST, _i32)]
    ).reshape(NW, CH, CHUNK)

    xp = jnp.pad(features, ((0, NP - N), (0, 0)))
    h0p = _mlp_tc(xp, W0, b0, W1, b1, W2, b2)

    zeros64 = jnp.zeros((RPT, NCLS), _f32)
    ones64 = jnp.ones((CHUNK, NCLS), _f32)
    onesf = jnp.ones((LANE,), _i32)
    zflag = jnp.zeros((16, LANE), _i32)

    hout, _, _, _, _ = _appnp_sc(h0p, srcp, dstp, zeros64, ones64, onesf, zflag)
    return hout[:N]
